# baseline (speedup 1.0000x reference)
import functools

import jax
import jax.numpy as jnp
from jax import lax
from jax.experimental import pallas as pl
from jax.experimental.pallas import tpu as pltpu
from jax.experimental.pallas import tpu_sc as plsc


def _gelu(x):
    return 0.5 * x * (1.0 + jax.lax.erf(x * 0.7071067811865476))


def _nav_kernel(last_ref, pW1_ref, pb1_ref, pW2_ref, pb2_ref,
                vW1_ref, vb1_ref, vW2_ref, vb2_ref,
                logits_ref, val_ref):
    h = last_ref[...]
    g1 = _gelu(jnp.dot(h, pW1_ref[...],
                       preferred_element_type=jnp.float32) + pb1_ref[...])
    logits_ref[...] = jnp.dot(g1, pW2_ref[...],
                              preferred_element_type=jnp.float32) + pb2_ref[...]
    v1 = _gelu(jnp.dot(h, vW1_ref[...],
                       preferred_element_type=jnp.float32) + vb1_ref[...])
    v = jnp.dot(v1, vW2_ref[...],
                preferred_element_type=jnp.float32) + vb2_ref[...]
    val_ref[...] = v.T


def _sc_router_kernel(lt_hbm, probs_hbm, act_hbm, lt_v, probs_s, act_s):
    wid = lax.axis_index("c") * 16 + lax.axis_index("s")

    pltpu.sync_copy(lt_hbm, lt_v)
    l0 = lt_v[0, :]
    l1 = lt_v[1, :]
    l2 = lt_v[2, :]
    l3 = lt_v[3, :]
    m = jnp.maximum(jnp.maximum(l0, l1), jnp.maximum(l2, l3))
    e0 = jnp.exp(l0 - m)
    e1 = jnp.exp(l1 - m)
    e2 = jnp.exp(l2 - m)
    e3 = jnp.exp(l3 - m)
    s = e0 + e1 + e2 + e3
    probs_s[0, :] = e0 / s
    probs_s[1, :] = e1 / s
    probs_s[2, :] = e2 / s
    probs_s[3, :] = e3 / s
    am = jnp.full((16,), 3, jnp.int32)
    am = jnp.where(l2 == m, jnp.full((16,), 2, jnp.int32), am)
    am = jnp.where(l1 == m, jnp.full((16,), 1, jnp.int32), am)
    am = jnp.where(l0 == m, jnp.full((16,), 0, jnp.int32), am)
    act_s[...] = am

    @pl.when(wid == 0)
    def _():
        pltpu.sync_copy(probs_s, probs_hbm)
        pltpu.sync_copy(act_s, act_hbm)


def _moe_kernel(sa_ref, x_ref, w1_ref, b1_ref, w2_ref, b2_ref,
                ow_ref, ob_ref, out_ref):
    del sa_ref
    x = x_ref[0]
    F = w1_ref.shape[2]
    FC = F // 2
    y = b2_ref[0].astype(jnp.float32)
    for c in range(2):
        sl = slice(c * FC, (c + 1) * FC)
        midc = _gelu(jnp.dot(x, w1_ref[0, :, sl],
                             preferred_element_type=jnp.float32)
                     + b1_ref[0, :, sl])
        y = y + jnp.dot(midc, w2_ref[0, sl, :],
                        preferred_element_type=jnp.float32)
    z = jnp.dot(y, ow_ref[...],
                preferred_element_type=jnp.float32) + ob_ref[...]
    out_ref[0] = x + 0.3 * z


def kernel(hidden_states, pW1, pb1, pW2, pb2, vW1, vb1, vW2, vb2,
           bW1, bb1, bW2, bb2, oW, ob):
    B, S, H = hidden_states.shape
    NB = pW2.shape[1]
    F = bW1.shape[2]
    TS = 512

    last = hidden_states[:, -1, :]
    logits, val2 = pl.pallas_call(
        _nav_kernel,
        out_shape=[
            jax.ShapeDtypeStruct((B, NB), jnp.float32),
            jax.ShapeDtypeStruct((1, B), jnp.float32),
        ],
    )(last, pW1, pb1.reshape(1, -1), pW2, pb2.reshape(1, -1),
      vW1, vb1.reshape(1, -1), vW2, vb2.reshape(1, 1))
    value = val2[0]

    lt16 = jnp.zeros((NB, 16), jnp.float32).at[:, :B].set(logits.T)

    mesh = plsc.VectorSubcoreMesh(core_axis_name="c", subcore_axis_name="s")
    sc_router = functools.partial(
        pl.kernel,
        mesh=mesh,
        out_type=[
            jax.ShapeDtypeStruct((NB, 16), jnp.float32),
            jax.ShapeDtypeStruct((16,), jnp.int32),
        ],
        scratch_types=[
            pltpu.VMEM((NB, 16), jnp.float32),
            pltpu.VMEM((NB, 16), jnp.float32),
            pltpu.VMEM((16,), jnp.int32),
        ],
    )(_sc_router_kernel)
    probs16, act16 = sc_router(lt16)
    probs = probs16[:, :B].T
    selected = act16[:B]

    grid_spec = pltpu.PrefetchScalarGridSpec(
        num_scalar_prefetch=1,
        grid=(B, S // TS),
        in_specs=[
            pl.BlockSpec((1, TS, H), lambda b, s, sa: (b, s, 0)),
            pl.BlockSpec((1, H, F), lambda b, s, sa: (sa[b], 0, 0)),
            pl.BlockSpec((1, 1, F), lambda b, s, sa: (sa[b], 0, 0)),
            pl.BlockSpec((1, F, H), lambda b, s, sa: (sa[b], 0, 0)),
            pl.BlockSpec((1, 1, H), lambda b, s, sa: (sa[b], 0, 0)),
            pl.BlockSpec(memory_space=pltpu.MemorySpace.VMEM),
            pl.BlockSpec(memory_space=pltpu.MemorySpace.VMEM),
        ],
        out_specs=pl.BlockSpec((1, TS, H), lambda b, s, sa: (b, s, 0)),
    )
    out = pl.pallas_call(
        _moe_kernel,
        grid_spec=grid_spec,
        out_shape=jax.ShapeDtypeStruct((B, S, H), jnp.float32),
        compiler_params=pltpu.CompilerParams(
            dimension_semantics=("arbitrary", "arbitrary")),
    )(selected, hidden_states, bW1, bb1.reshape(NB, 1, F), bW2,
      bb2.reshape(NB, 1, H), oW, ob.reshape(1, -1))

    return (out, probs, selected, value)


# speedup vs baseline: 1.0006x; 1.0006x over previous
import functools

import jax
import jax.numpy as jnp
from jax import lax
from jax.experimental import pallas as pl
from jax.experimental.pallas import tpu as pltpu
from jax.experimental.pallas import tpu_sc as plsc


def _gelu(x):
    return 0.5 * x * (1.0 + jax.lax.erf(x * 0.7071067811865476))


def _nav_kernel(last_ref, pW1_ref, pb1_ref, pW2_ref, pb2_ref,
                vW1_ref, vb1_ref, vW2_ref, vb2_ref,
                logits_ref, act_ref, val_ref):
    h = last_ref[...]
    g1 = _gelu(jnp.dot(h, pW1_ref[...],
                       preferred_element_type=jnp.float32) + pb1_ref[...])
    logits = jnp.dot(g1, pW2_ref[...],
                     preferred_element_type=jnp.float32) + pb2_ref[...]
    logits_ref[...] = logits
    act_ref[...] = jnp.argmax(logits, axis=-1)[None, :].astype(jnp.int32)
    v1 = _gelu(jnp.dot(h, vW1_ref[...],
                       preferred_element_type=jnp.float32) + vb1_ref[...])
    v = jnp.dot(v1, vW2_ref[...],
                preferred_element_type=jnp.float32) + vb2_ref[...]
    val_ref[...] = v.T


def _sc_router_kernel(lt_hbm, probs_hbm, lt_v, probs_s):
    wid = lax.axis_index("c") * 16 + lax.axis_index("s")

    pltpu.sync_copy(lt_hbm, lt_v)
    l0 = lt_v[0, :]
    l1 = lt_v[1, :]
    l2 = lt_v[2, :]
    l3 = lt_v[3, :]
    m = jnp.maximum(jnp.maximum(l0, l1), jnp.maximum(l2, l3))
    e0 = jnp.exp(l0 - m)
    e1 = jnp.exp(l1 - m)
    e2 = jnp.exp(l2 - m)
    e3 = jnp.exp(l3 - m)
    s = e0 + e1 + e2 + e3
    probs_s[0, :] = e0 / s
    probs_s[1, :] = e1 / s
    probs_s[2, :] = e2 / s
    probs_s[3, :] = e3 / s

    @pl.when(wid == 0)
    def _():
        pltpu.sync_copy(probs_s, probs_hbm)


def _moe_kernel(sa_ref, x_ref, w1_ref, b1_ref, w2_ref, b2_ref,
                ow_ref, ob_ref, out_ref):
    del sa_ref
    x = x_ref[0]
    F = w1_ref.shape[2]
    FC = F // 2
    y = b2_ref[0].astype(jnp.float32)
    for c in range(2):
        sl = slice(c * FC, (c + 1) * FC)
        midc = _gelu(jnp.dot(x, w1_ref[0, :, sl],
                             preferred_element_type=jnp.float32)
                     + b1_ref[0, :, sl])
        y = y + jnp.dot(midc, w2_ref[0, sl, :],
                        preferred_element_type=jnp.float32)
    z = jnp.dot(y, ow_ref[...],
                preferred_element_type=jnp.float32) + ob_ref[...]
    out_ref[0] = x + 0.3 * z


def kernel(hidden_states, pW1, pb1, pW2, pb2, vW1, vb1, vW2, vb2,
           bW1, bb1, bW2, bb2, oW, ob):
    B, S, H = hidden_states.shape
    NB = pW2.shape[1]
    F = bW1.shape[2]
    TS = 512

    last = hidden_states[:, -1, :]
    logits, act2, val2 = pl.pallas_call(
        _nav_kernel,
        out_shape=[
            jax.ShapeDtypeStruct((B, NB), jnp.float32),
            jax.ShapeDtypeStruct((1, B), jnp.int32),
            jax.ShapeDtypeStruct((1, B), jnp.float32),
        ],
    )(last, pW1, pb1.reshape(1, -1), pW2, pb2.reshape(1, -1),
      vW1, vb1.reshape(1, -1), vW2, vb2.reshape(1, 1))
    value = val2[0]
    selected = act2[0]

    lt16 = jnp.zeros((NB, 16), jnp.float32).at[:, :B].set(logits.T)

    mesh = plsc.VectorSubcoreMesh(core_axis_name="c", subcore_axis_name="s")
    sc_router = functools.partial(
        pl.kernel,
        mesh=mesh,
        out_type=[
            jax.ShapeDtypeStruct((NB, 16), jnp.float32),
        ],
        scratch_types=[
            pltpu.VMEM((NB, 16), jnp.float32),
            pltpu.VMEM((NB, 16), jnp.float32),
        ],
    )(_sc_router_kernel)
    (probs16,) = sc_router(lt16)
    probs = probs16[:, :B].T

    grid_spec = pltpu.PrefetchScalarGridSpec(
        num_scalar_prefetch=1,
        grid=(B, S // TS),
        in_specs=[
            pl.BlockSpec((1, TS, H), lambda b, s, sa: (b, s, 0)),
            pl.BlockSpec((1, H, F), lambda b, s, sa: (sa[b], 0, 0)),
            pl.BlockSpec((1, 1, F), lambda b, s, sa: (sa[b], 0, 0)),
            pl.BlockSpec((1, F, H), lambda b, s, sa: (sa[b], 0, 0)),
            pl.BlockSpec((1, 1, H), lambda b, s, sa: (sa[b], 0, 0)),
            pl.BlockSpec(memory_space=pltpu.MemorySpace.VMEM),
            pl.BlockSpec(memory_space=pltpu.MemorySpace.VMEM),
        ],
        out_specs=pl.BlockSpec((1, TS, H), lambda b, s, sa: (b, s, 0)),
    )
    out = pl.pallas_call(
        _moe_kernel,
        grid_spec=grid_spec,
        out_shape=jax.ShapeDtypeStruct((B, S, H), jnp.float32),
        compiler_params=pltpu.CompilerParams(
            dimension_semantics=("arbitrary", "arbitrary")),
    )(selected, hidden_states, bW1, bb1.reshape(NB, 1, F), bW2,
      bb2.reshape(NB, 1, H), oW, ob.reshape(1, -1))

    return (out, probs, selected, value)


# speedup vs baseline: 1.0145x; 1.0139x over previous
import functools

import jax
import jax.numpy as jnp
from jax import lax
from jax.experimental import pallas as pl
from jax.experimental.pallas import tpu as pltpu
from jax.experimental.pallas import tpu_sc as plsc


def _gelu(x):
    return 0.5 * x * (1.0 + jax.lax.erf(x * 0.7071067811865476))


def _nav_kernel(last_ref, pW1_ref, pb1_ref, pW2T_ref, pb2T_ref,
                vW1_ref, vb1_ref, vW2_ref, vb2_ref,
                lt_ref, val_ref):
    h = last_ref[...]
    B = h.shape[0]
    g1 = _gelu(jnp.dot(h, pW1_ref[...],
                       preferred_element_type=jnp.float32) + pb1_ref[...])
    logits_t = lax.dot_general(
        pW2T_ref[...], g1, (((1,), (1,)), ((), ())),
        preferred_element_type=jnp.float32) + pb2T_ref[...]
    lt_ref[...] = jnp.concatenate(
        [logits_t, jnp.zeros((logits_t.shape[0], 16 - B), jnp.float32)],
        axis=1)
    v1 = _gelu(jnp.dot(h, vW1_ref[...],
                       preferred_element_type=jnp.float32) + vb1_ref[...])
    v = jnp.dot(v1, vW2_ref[...],
                preferred_element_type=jnp.float32) + vb2_ref[...]
    val_ref[...] = v.T


def _sc_router_kernel(lt_hbm, probs_hbm, act_hbm, lt_v, probs_s, act_s):
    wid = lax.axis_index("c") * 16 + lax.axis_index("s")

    pltpu.sync_copy(lt_hbm, lt_v)
    l0 = lt_v[0, :]
    l1 = lt_v[1, :]
    l2 = lt_v[2, :]
    l3 = lt_v[3, :]
    m = jnp.maximum(jnp.maximum(l0, l1), jnp.maximum(l2, l3))
    e0 = jnp.exp(l0 - m)
    e1 = jnp.exp(l1 - m)
    e2 = jnp.exp(l2 - m)
    e3 = jnp.exp(l3 - m)
    s = e0 + e1 + e2 + e3
    probs_s[0, :] = e0 / s
    probs_s[1, :] = e1 / s
    probs_s[2, :] = e2 / s
    probs_s[3, :] = e3 / s
    am = jnp.full((16,), 3, jnp.int32)
    am = jnp.where(l2 == m, jnp.full((16,), 2, jnp.int32), am)
    am = jnp.where(l1 == m, jnp.full((16,), 1, jnp.int32), am)
    am = jnp.where(l0 == m, jnp.full((16,), 0, jnp.int32), am)
    act_s[...] = am

    @pl.when(wid == 0)
    def _():
        pltpu.sync_copy(probs_s, probs_hbm)
        pltpu.sync_copy(act_s, act_hbm)


def _moe_kernel(sa_ref, x_ref, w1_ref, b1_ref, w2_ref, b2_ref,
                ow_ref, ob_ref, out_ref):
    del sa_ref
    x = x_ref[0]
    F = w1_ref.shape[2]
    FC = F // 2
    y = b2_ref[0].astype(jnp.float32)
    for c in range(2):
        sl = slice(c * FC, (c + 1) * FC)
        midc = _gelu(jnp.dot(x, w1_ref[0, :, sl],
                             preferred_element_type=jnp.float32)
                     + b1_ref[0, :, sl])
        y = y + jnp.dot(midc, w2_ref[0, sl, :],
                        preferred_element_type=jnp.float32)
    z = jnp.dot(y, ow_ref[...],
                preferred_element_type=jnp.float32) + ob_ref[...]
    out_ref[0] = x + 0.3 * z


def kernel(hidden_states, pW1, pb1, pW2, pb2, vW1, vb1, vW2, vb2,
           bW1, bb1, bW2, bb2, oW, ob):
    B, S, H = hidden_states.shape
    NB = pW2.shape[1]
    F = bW1.shape[2]
    TS = 512

    last = hidden_states[:, -1, :]
    lt16, val2 = pl.pallas_call(
        _nav_kernel,
        out_shape=[
            jax.ShapeDtypeStruct((NB, 16), jnp.float32),
            jax.ShapeDtypeStruct((1, B), jnp.float32),
        ],
    )(last, pW1, pb1.reshape(1, -1), pW2.T, pb2.reshape(-1, 1),
      vW1, vb1.reshape(1, -1), vW2, vb2.reshape(1, 1))
    value = val2[0]

    mesh = plsc.VectorSubcoreMesh(core_axis_name="c", subcore_axis_name="s")
    sc_router = functools.partial(
        pl.kernel,
        mesh=mesh,
        out_type=[
            jax.ShapeDtypeStruct((NB, 16), jnp.float32),
            jax.ShapeDtypeStruct((16,), jnp.int32),
        ],
        scratch_types=[
            pltpu.VMEM((NB, 16), jnp.float32),
            pltpu.VMEM((NB, 16), jnp.float32),
            pltpu.VMEM((16,), jnp.int32),
        ],
    )(_sc_router_kernel)
    probs16, act16 = sc_router(lt16)
    probs = probs16[:, :B].T
    selected = act16[:B]

    grid_spec = pltpu.PrefetchScalarGridSpec(
        num_scalar_prefetch=1,
        grid=(B, S // TS),
        in_specs=[
            pl.BlockSpec((1, TS, H), lambda b, s, sa: (b, s, 0)),
            pl.BlockSpec((1, H, F), lambda b, s, sa: (sa[b], 0, 0)),
            pl.BlockSpec((1, 1, F), lambda b, s, sa: (sa[b], 0, 0)),
            pl.BlockSpec((1, F, H), lambda b, s, sa: (sa[b], 0, 0)),
            pl.BlockSpec((1, 1, H), lambda b, s, sa: (sa[b], 0, 0)),
            pl.BlockSpec(memory_space=pltpu.MemorySpace.VMEM),
            pl.BlockSpec(memory_space=pltpu.MemorySpace.VMEM),
        ],
        out_specs=pl.BlockSpec((1, TS, H), lambda b, s, sa: (b, s, 0)),
    )
    out = pl.pallas_call(
        _moe_kernel,
        grid_spec=grid_spec,
        out_shape=jax.ShapeDtypeStruct((B, S, H), jnp.float32),
        compiler_params=pltpu.CompilerParams(
            dimension_semantics=("arbitrary", "arbitrary")),
    )(act16, hidden_states, bW1, bb1.reshape(NB, 1, F), bW2,
      bb2.reshape(NB, 1, H), oW, ob.reshape(1, -1))

    return (out, probs, selected, value)


# speedup vs baseline: 1.0226x; 1.0080x over previous
import functools

import jax
import jax.numpy as jnp
from jax import lax
from jax.experimental import pallas as pl
from jax.experimental.pallas import tpu as pltpu
from jax.experimental.pallas import tpu_sc as plsc


def _gelu(x):
    return 0.5 * x * (1.0 + jax.lax.erf(x * 0.7071067811865476))


def _nav_kernel(last_ref, pW1_ref, pb1_ref, pW2T_ref, pb2T_ref,
                vW1_ref, vb1_ref, vW2_ref, vb2_ref,
                lt_ref, val_ref):
    h = last_ref[...]
    B = h.shape[0]
    g1 = _gelu(jnp.dot(h, pW1_ref[...],
                       preferred_element_type=jnp.float32) + pb1_ref[...])
    logits_t = lax.dot_general(
        pW2T_ref[...], g1, (((1,), (1,)), ((), ())),
        preferred_element_type=jnp.float32) + pb2T_ref[...]
    lt_ref[...] = jnp.concatenate(
        [logits_t, jnp.zeros((logits_t.shape[0], 16 - B), jnp.float32)],
        axis=1)
    v1 = _gelu(jnp.dot(h, vW1_ref[...],
                       preferred_element_type=jnp.float32) + vb1_ref[...])
    v = jnp.dot(v1, vW2_ref[...],
                preferred_element_type=jnp.float32) + vb2_ref[...]
    val_ref[...] = v.T


def _sc_router_kernel(lt_hbm, probs_hbm, act_hbm, lt_v, probs_s, act_s):
    wid = lax.axis_index("c") * 16 + lax.axis_index("s")

    pltpu.sync_copy(lt_hbm, lt_v)
    l0 = lt_v[0, :]
    l1 = lt_v[1, :]
    l2 = lt_v[2, :]
    l3 = lt_v[3, :]
    m = jnp.maximum(jnp.maximum(l0, l1), jnp.maximum(l2, l3))
    e0 = jnp.exp(l0 - m)
    e1 = jnp.exp(l1 - m)
    e2 = jnp.exp(l2 - m)
    e3 = jnp.exp(l3 - m)
    s = e0 + e1 + e2 + e3
    probs_s[0, :] = e0 / s
    probs_s[1, :] = e1 / s
    probs_s[2, :] = e2 / s
    probs_s[3, :] = e3 / s
    am = jnp.full((16,), 3, jnp.int32)
    am = jnp.where(l2 == m, jnp.full((16,), 2, jnp.int32), am)
    am = jnp.where(l1 == m, jnp.full((16,), 1, jnp.int32), am)
    am = jnp.where(l0 == m, jnp.full((16,), 0, jnp.int32), am)
    act_s[...] = am

    @pl.when(wid == 0)
    def _():
        pltpu.sync_copy(probs_s, probs_hbm)
        pltpu.sync_copy(act_s, act_hbm)


def _moe_kernel(sa_ref, x_ref, w1_ref, b1_ref, w2_ref, b2_ref,
                ow_ref, ob_ref, out_ref):
    del sa_ref
    x = x_ref[0]
    mid = _gelu(jnp.dot(x, w1_ref[0],
                        preferred_element_type=jnp.float32) + b1_ref[0])
    y = jnp.dot(mid, w2_ref[0],
                preferred_element_type=jnp.float32) + b2_ref[0]
    z = jnp.dot(y, ow_ref[...],
                preferred_element_type=jnp.float32) + ob_ref[...]
    out_ref[0] = x + 0.3 * z


def kernel(hidden_states, pW1, pb1, pW2, pb2, vW1, vb1, vW2, vb2,
           bW1, bb1, bW2, bb2, oW, ob):
    B, S, H = hidden_states.shape
    NB = pW2.shape[1]
    F = bW1.shape[2]
    TS = 512

    last = hidden_states[:, -1, :]
    lt16, val2 = pl.pallas_call(
        _nav_kernel,
        out_shape=[
            jax.ShapeDtypeStruct((NB, 16), jnp.float32),
            jax.ShapeDtypeStruct((1, B), jnp.float32),
        ],
    )(last, pW1, pb1.reshape(1, -1), pW2.T, pb2.reshape(-1, 1),
      vW1, vb1.reshape(1, -1), vW2, vb2.reshape(1, 1))
    value = val2[0]

    mesh = plsc.VectorSubcoreMesh(core_axis_name="c", subcore_axis_name="s")
    sc_router = functools.partial(
        pl.kernel,
        mesh=mesh,
        out_type=[
            jax.ShapeDtypeStruct((NB, 16), jnp.float32),
            jax.ShapeDtypeStruct((16,), jnp.int32),
        ],
        scratch_types=[
            pltpu.VMEM((NB, 16), jnp.float32),
            pltpu.VMEM((NB, 16), jnp.float32),
            pltpu.VMEM((16,), jnp.int32),
        ],
    )(_sc_router_kernel)
    probs16, act16 = sc_router(lt16)
    probs = probs16[:, :B].T
    selected = act16[:B]

    grid_spec = pltpu.PrefetchScalarGridSpec(
        num_scalar_prefetch=1,
        grid=(B, S // TS),
        in_specs=[
            pl.BlockSpec((1, TS, H), lambda b, s, sa: (b, s, 0)),
            pl.BlockSpec((1, H, F), lambda b, s, sa: (sa[b], 0, 0)),
            pl.BlockSpec((1, 1, F), lambda b, s, sa: (sa[b], 0, 0)),
            pl.BlockSpec((1, F, H), lambda b, s, sa: (sa[b], 0, 0)),
            pl.BlockSpec((1, 1, H), lambda b, s, sa: (sa[b], 0, 0)),
            pl.BlockSpec(memory_space=pltpu.MemorySpace.VMEM),
            pl.BlockSpec(memory_space=pltpu.MemorySpace.VMEM),
        ],
        out_specs=pl.BlockSpec((1, TS, H), lambda b, s, sa: (b, s, 0)),
    )
    out = pl.pallas_call(
        _moe_kernel,
        grid_spec=grid_spec,
        out_shape=jax.ShapeDtypeStruct((B, S, H), jnp.float32),
        compiler_params=pltpu.CompilerParams(
            dimension_semantics=("arbitrary", "arbitrary")),
    )(act16, hidden_states, bW1, bb1.reshape(NB, 1, F), bW2,
      bb2.reshape(NB, 1, H), oW, ob.reshape(1, -1))

    return (out, probs, selected, value)


# speedup vs baseline: 1.0258x; 1.0031x over previous
import functools

import jax
import jax.numpy as jnp
from jax import lax
from jax.experimental import pallas as pl
from jax.experimental.pallas import tpu as pltpu
from jax.experimental.pallas import tpu_sc as plsc


def _gelu(x):
    return 0.5 * x * (1.0 + jax.lax.erf(x * 0.7071067811865476))


def _nav_kernel(last_ref, pW1_ref, pb1_ref, pW2T_ref, pb2T_ref,
                vW1_ref, vb1_ref, vW2_ref, vb2_ref,
                lt_ref, val_ref):
    h = last_ref[...]
    B = h.shape[0]
    g1 = _gelu(jnp.dot(h, pW1_ref[...],
                       preferred_element_type=jnp.float32) + pb1_ref[...])
    logits_t = lax.dot_general(
        pW2T_ref[...], g1, (((1,), (1,)), ((), ())),
        preferred_element_type=jnp.float32) + pb2T_ref[...]
    lt_ref[...] = jnp.concatenate(
        [logits_t, jnp.zeros((logits_t.shape[0], 16 - B), jnp.float32)],
        axis=1)
    v1 = _gelu(jnp.dot(h, vW1_ref[...],
                       preferred_element_type=jnp.float32) + vb1_ref[...])
    v = jnp.dot(v1, vW2_ref[...],
                preferred_element_type=jnp.float32) + vb2_ref[...]
    val_ref[...] = v.T


def _sc_router_kernel(lt_hbm, probs_hbm, act_hbm, lt_v, probs_s, act_s):
    wid = lax.axis_index("c") * 16 + lax.axis_index("s")

    pltpu.sync_copy(lt_hbm, lt_v)
    l0 = lt_v[0, :]
    l1 = lt_v[1, :]
    l2 = lt_v[2, :]
    l3 = lt_v[3, :]
    m = jnp.maximum(jnp.maximum(l0, l1), jnp.maximum(l2, l3))
    e0 = jnp.exp(l0 - m)
    e1 = jnp.exp(l1 - m)
    e2 = jnp.exp(l2 - m)
    e3 = jnp.exp(l3 - m)
    s = e0 + e1 + e2 + e3
    probs_s[0, :] = e0 / s
    probs_s[1, :] = e1 / s
    probs_s[2, :] = e2 / s
    probs_s[3, :] = e3 / s
    am = jnp.full((16,), 3, jnp.int32)
    am = jnp.where(l2 == m, jnp.full((16,), 2, jnp.int32), am)
    am = jnp.where(l1 == m, jnp.full((16,), 1, jnp.int32), am)
    am = jnp.where(l0 == m, jnp.full((16,), 0, jnp.int32), am)
    act_s[...] = am

    @pl.when(wid == 0)
    def _():
        pltpu.sync_copy(probs_s, probs_hbm)
        pltpu.sync_copy(act_s, act_hbm)


def _moe_kernel(sa_ref, x_ref, w1_ref, b1_ref, w2_ref, b2_ref,
                ow_ref, ob_ref, out_ref):
    del sa_ref
    x = x_ref[0]
    F = w1_ref.shape[2]
    FC = F // 4
    y = b2_ref[0].astype(jnp.float32)
    for c in range(4):
        sl = slice(c * FC, (c + 1) * FC)
        midc = _gelu(jnp.dot(x, w1_ref[0, :, sl],
                             preferred_element_type=jnp.float32)
                     + b1_ref[0, :, sl])
        y = y + jnp.dot(midc, w2_ref[0, sl, :],
                        preferred_element_type=jnp.float32)
    z = jnp.dot(y, ow_ref[...],
                preferred_element_type=jnp.float32) + ob_ref[...]
    out_ref[0] = x + 0.3 * z


def kernel(hidden_states, pW1, pb1, pW2, pb2, vW1, vb1, vW2, vb2,
           bW1, bb1, bW2, bb2, oW, ob):
    B, S, H = hidden_states.shape
    NB = pW2.shape[1]
    F = bW1.shape[2]
    TS = 1024

    last = hidden_states[:, -1, :]
    lt16, val2 = pl.pallas_call(
        _nav_kernel,
        out_shape=[
            jax.ShapeDtypeStruct((NB, 16), jnp.float32),
            jax.ShapeDtypeStruct((1, B), jnp.float32),
        ],
    )(last, pW1, pb1.reshape(1, -1), pW2.T, pb2.reshape(-1, 1),
      vW1, vb1.reshape(1, -1), vW2, vb2.reshape(1, 1))
    value = val2[0]

    mesh = plsc.VectorSubcoreMesh(core_axis_name="c", subcore_axis_name="s")
    sc_router = functools.partial(
        pl.kernel,
        mesh=mesh,
        out_type=[
            jax.ShapeDtypeStruct((NB, 16), jnp.float32),
            jax.ShapeDtypeStruct((16,), jnp.int32),
        ],
        scratch_types=[
            pltpu.VMEM((NB, 16), jnp.float32),
            pltpu.VMEM((NB, 16), jnp.float32),
            pltpu.VMEM((16,), jnp.int32),
        ],
    )(_sc_router_kernel)
    probs16, act16 = sc_router(lt16)
    probs = probs16[:, :B].T
    selected = act16[:B]

    grid_spec = pltpu.PrefetchScalarGridSpec(
        num_scalar_prefetch=1,
        grid=(B, S // TS),
        in_specs=[
            pl.BlockSpec((1, TS, H), lambda b, s, sa: (b, s, 0)),
            pl.BlockSpec((1, H, F), lambda b, s, sa: (sa[b], 0, 0)),
            pl.BlockSpec((1, 1, F), lambda b, s, sa: (sa[b], 0, 0)),
            pl.BlockSpec((1, F, H), lambda b, s, sa: (sa[b], 0, 0)),
            pl.BlockSpec((1, 1, H), lambda b, s, sa: (sa[b], 0, 0)),
            pl.BlockSpec(memory_space=pltpu.MemorySpace.VMEM),
            pl.BlockSpec(memory_space=pltpu.MemorySpace.VMEM),
        ],
        out_specs=pl.BlockSpec((1, TS, H), lambda b, s, sa: (b, s, 0)),
    )
    out = pl.pallas_call(
        _moe_kernel,
        grid_spec=grid_spec,
        out_shape=jax.ShapeDtypeStruct((B, S, H), jnp.float32),
        compiler_params=pltpu.CompilerParams(
            dimension_semantics=("arbitrary", "arbitrary")),
    )(act16, hidden_states, bW1, bb1.reshape(NB, 1, F), bW2,
      bb2.reshape(NB, 1, H), oW, ob.reshape(1, -1))

    return (out, probs, selected, value)


# speedup vs baseline: 1.0311x; 1.0051x over previous
import functools

import jax
import jax.numpy as jnp
from jax import lax
from jax.experimental import pallas as pl
from jax.experimental.pallas import tpu as pltpu
from jax.experimental.pallas import tpu_sc as plsc


def _gelu(x):
    return 0.5 * x * (1.0 + jax.lax.erf(x * 0.7071067811865476))


def _nav_kernel(last_ref, pW1_ref, pb1_ref, pW2T_ref, pb2T_ref,
                vW1_ref, vb1_ref, vW2_ref, vb2_ref,
                lt_ref, val_ref):
    h = last_ref[...]
    B = h.shape[0]
    g1 = _gelu(jnp.dot(h, pW1_ref[...],
                       preferred_element_type=jnp.float32) + pb1_ref[...])
    logits_t = lax.dot_general(
        pW2T_ref[...], g1, (((1,), (1,)), ((), ())),
        preferred_element_type=jnp.float32) + pb2T_ref[...]
    lt_ref[...] = jnp.concatenate(
        [logits_t, jnp.zeros((logits_t.shape[0], 16 - B), jnp.float32)],
        axis=1)
    v1 = _gelu(jnp.dot(h, vW1_ref[...],
                       preferred_element_type=jnp.float32) + vb1_ref[...])
    v = jnp.dot(v1, vW2_ref[...],
                preferred_element_type=jnp.float32) + vb2_ref[...]
    val_ref[...] = v.T


def _sc_router_kernel(lt_hbm, probs_hbm, act_hbm, lt_v, probs_s, act_s):
    wid = lax.axis_index("c") * 16 + lax.axis_index("s")

    pltpu.sync_copy(lt_hbm, lt_v)
    l0 = lt_v[0, :]
    l1 = lt_v[1, :]
    l2 = lt_v[2, :]
    l3 = lt_v[3, :]
    m = jnp.maximum(jnp.maximum(l0, l1), jnp.maximum(l2, l3))
    e0 = jnp.exp(l0 - m)
    e1 = jnp.exp(l1 - m)
    e2 = jnp.exp(l2 - m)
    e3 = jnp.exp(l3 - m)
    s = e0 + e1 + e2 + e3
    probs_s[0, :] = e0 / s
    probs_s[1, :] = e1 / s
    probs_s[2, :] = e2 / s
    probs_s[3, :] = e3 / s
    am = jnp.full((16,), 3, jnp.int32)
    am = jnp.where(l2 == m, jnp.full((16,), 2, jnp.int32), am)
    am = jnp.where(l1 == m, jnp.full((16,), 1, jnp.int32), am)
    am = jnp.where(l0 == m, jnp.full((16,), 0, jnp.int32), am)
    act_s[...] = am

    @pl.when(wid == 0)
    def _():
        pltpu.sync_copy(probs_s, probs_hbm)
        pltpu.sync_copy(act_s, act_hbm)


def _moe_kernel(sa_ref, x_ref, w1_ref, b1_ref, w2_ref, b2_ref,
                ow_ref, ob_ref, out_ref):
    del sa_ref
    TS = x_ref.shape[1]
    F = w1_ref.shape[2]
    FC = F // 4
    RC = TS // 2
    for r in range(2):
        x = x_ref[0, pl.ds(r * RC, RC)]
        y = b2_ref[0].astype(jnp.float32)
        for c in range(4):
            sl = slice(c * FC, (c + 1) * FC)
            midc = _gelu(jnp.dot(x, w1_ref[0, :, sl],
                                 preferred_element_type=jnp.float32)
                         + b1_ref[0, :, sl])
            y = y + jnp.dot(midc, w2_ref[0, sl, :],
                            preferred_element_type=jnp.float32)
        z = jnp.dot(y, ow_ref[...],
                    preferred_element_type=jnp.float32) + ob_ref[...]
        out_ref[0, pl.ds(r * RC, RC)] = x + 0.3 * z


def kernel(hidden_states, pW1, pb1, pW2, pb2, vW1, vb1, vW2, vb2,
           bW1, bb1, bW2, bb2, oW, ob):
    B, S, H = hidden_states.shape
    NB = pW2.shape[1]
    F = bW1.shape[2]
    TS = 1024

    last = hidden_states[:, -1, :]
    lt16, val2 = pl.pallas_call(
        _nav_kernel,
        out_shape=[
            jax.ShapeDtypeStruct((NB, 16), jnp.float32),
            jax.ShapeDtypeStruct((1, B), jnp.float32),
        ],
    )(last, pW1, pb1.reshape(1, -1), pW2.T, pb2.reshape(-1, 1),
      vW1, vb1.reshape(1, -1), vW2, vb2.reshape(1, 1))
    value = val2[0]

    mesh = plsc.VectorSubcoreMesh(core_axis_name="c", subcore_axis_name="s")
    sc_router = functools.partial(
        pl.kernel,
        mesh=mesh,
        out_type=[
            jax.ShapeDtypeStruct((NB, 16), jnp.float32),
            jax.ShapeDtypeStruct((16,), jnp.int32),
        ],
        scratch_types=[
            pltpu.VMEM((NB, 16), jnp.float32),
            pltpu.VMEM((NB, 16), jnp.float32),
            pltpu.VMEM((16,), jnp.int32),
        ],
    )(_sc_router_kernel)
    probs16, act16 = sc_router(lt16)
    probs = probs16[:, :B].T
    selected = act16[:B]

    grid_spec = pltpu.PrefetchScalarGridSpec(
        num_scalar_prefetch=1,
        grid=(B, S // TS),
        in_specs=[
            pl.BlockSpec((1, TS, H), lambda b, s, sa: (b, s, 0)),
            pl.BlockSpec((1, H, F), lambda b, s, sa: (sa[b], 0, 0)),
            pl.BlockSpec((1, 1, F), lambda b, s, sa: (sa[b], 0, 0)),
            pl.BlockSpec((1, F, H), lambda b, s, sa: (sa[b], 0, 0)),
            pl.BlockSpec((1, 1, H), lambda b, s, sa: (sa[b], 0, 0)),
            pl.BlockSpec(memory_space=pltpu.MemorySpace.VMEM),
            pl.BlockSpec(memory_space=pltpu.MemorySpace.VMEM),
        ],
        out_specs=pl.BlockSpec((1, TS, H), lambda b, s, sa: (b, s, 0)),
    )
    out = pl.pallas_call(
        _moe_kernel,
        grid_spec=grid_spec,
        out_shape=jax.ShapeDtypeStruct((B, S, H), jnp.float32),
        compiler_params=pltpu.CompilerParams(
            dimension_semantics=("arbitrary", "arbitrary")),
    )(act16, hidden_states, bW1, bb1.reshape(NB, 1, F), bW2,
      bb2.reshape(NB, 1, H), oW, ob.reshape(1, -1))

    return (out, probs, selected, value)


# speedup vs baseline: 1.0503x; 1.0187x over previous
import functools

import jax
import jax.numpy as jnp
from jax import lax
from jax.experimental import pallas as pl
from jax.experimental.pallas import tpu as pltpu
from jax.experimental.pallas import tpu_sc as plsc


def _gelu(x):
    return 0.5 * x * (1.0 + jax.lax.erf(x * 0.7071067811865476))


def _nav_kernel(last_ref, pW1_ref, pb1_ref, pW2T_ref, pb2T_ref,
                vW1_ref, vb1_ref, vW2_ref, vb2_ref,
                lt_ref, val_ref):
    h = last_ref[...]
    B = h.shape[0]
    g1 = _gelu(jnp.dot(h, pW1_ref[...],
                       preferred_element_type=jnp.float32) + pb1_ref[...])
    logits_t = lax.dot_general(
        pW2T_ref[...], g1, (((1,), (1,)), ((), ())),
        preferred_element_type=jnp.float32) + pb2T_ref[...]
    lt_ref[...] = jnp.concatenate(
        [logits_t, jnp.zeros((logits_t.shape[0], 16 - B), jnp.float32)],
        axis=1)
    v1 = _gelu(jnp.dot(h, vW1_ref[...],
                       preferred_element_type=jnp.float32) + vb1_ref[...])
    v = jnp.dot(v1, vW2_ref[...],
                preferred_element_type=jnp.float32) + vb2_ref[...]
    val_ref[...] = v.T


def _sc_router_kernel(lt_hbm, probs_hbm, act_hbm, lt_v, probs_s, act_s):
    wid = lax.axis_index("c") * 16 + lax.axis_index("s")

    pltpu.sync_copy(lt_hbm, lt_v)
    l0 = lt_v[0, :]
    l1 = lt_v[1, :]
    l2 = lt_v[2, :]
    l3 = lt_v[3, :]
    m = jnp.maximum(jnp.maximum(l0, l1), jnp.maximum(l2, l3))
    e0 = jnp.exp(l0 - m)
    e1 = jnp.exp(l1 - m)
    e2 = jnp.exp(l2 - m)
    e3 = jnp.exp(l3 - m)
    s = e0 + e1 + e2 + e3
    probs_s[0, :] = e0 / s
    probs_s[1, :] = e1 / s
    probs_s[2, :] = e2 / s
    probs_s[3, :] = e3 / s
    am = jnp.full((16,), 3, jnp.int32)
    am = jnp.where(l2 == m, jnp.full((16,), 2, jnp.int32), am)
    am = jnp.where(l1 == m, jnp.full((16,), 1, jnp.int32), am)
    am = jnp.where(l0 == m, jnp.full((16,), 0, jnp.int32), am)
    act_s[...] = am

    @pl.when(wid == 0)
    def _():
        pltpu.sync_copy(probs_s, probs_hbm)
        pltpu.sync_copy(act_s, act_hbm)


def _moe_kernel(sa_ref, x_ref, w1_ref, b1_ref, w2_ref, b2_ref,
                ow_ref, ob_ref, out_ref):
    del sa_ref
    TS = x_ref.shape[1]
    F = w1_ref.shape[2]
    FC = F // 2
    RC = TS // 2
    for r in range(2):
        x = x_ref[0, pl.ds(r * RC, RC)]
        y = b2_ref[0].astype(jnp.float32)
        for c in range(2):
            sl = slice(c * FC, (c + 1) * FC)
            midc = _gelu(jnp.dot(x, w1_ref[0, :, sl],
                                 preferred_element_type=jnp.float32)
                         + b1_ref[0, :, sl])
            y = y + jnp.dot(midc, w2_ref[0, sl, :],
                            preferred_element_type=jnp.float32)
        z = jnp.dot(y, ow_ref[...],
                    preferred_element_type=jnp.float32) + ob_ref[...]
        out_ref[0, pl.ds(r * RC, RC)] = x + 0.3 * z


def kernel(hidden_states, pW1, pb1, pW2, pb2, vW1, vb1, vW2, vb2,
           bW1, bb1, bW2, bb2, oW, ob):
    B, S, H = hidden_states.shape
    NB = pW2.shape[1]
    F = bW1.shape[2]
    TS = 1024

    last = hidden_states[:, -1, :]
    lt16, val2 = pl.pallas_call(
        _nav_kernel,
        out_shape=[
            jax.ShapeDtypeStruct((NB, 16), jnp.float32),
            jax.ShapeDtypeStruct((1, B), jnp.float32),
        ],
    )(last, pW1, pb1.reshape(1, -1), pW2.T, pb2.reshape(-1, 1),
      vW1, vb1.reshape(1, -1), vW2, vb2.reshape(1, 1))
    value = val2[0]

    mesh = plsc.VectorSubcoreMesh(core_axis_name="c", subcore_axis_name="s")
    sc_router = functools.partial(
        pl.kernel,
        mesh=mesh,
        out_type=[
            jax.ShapeDtypeStruct((NB, 16), jnp.float32),
            jax.ShapeDtypeStruct((16,), jnp.int32),
        ],
        scratch_types=[
            pltpu.VMEM((NB, 16), jnp.float32),
            pltpu.VMEM((NB, 16), jnp.float32),
            pltpu.VMEM((16,), jnp.int32),
        ],
    )(_sc_router_kernel)
    probs16, act16 = sc_router(lt16)
    probs = probs16[:, :B].T
    selected = act16[:B]

    grid_spec = pltpu.PrefetchScalarGridSpec(
        num_scalar_prefetch=1,
        grid=(B, S // TS),
        in_specs=[
            pl.BlockSpec((1, TS, H), lambda b, s, sa: (b, s, 0)),
            pl.BlockSpec((1, H, F), lambda b, s, sa: (sa[b], 0, 0)),
            pl.BlockSpec((1, 1, F), lambda b, s, sa: (sa[b], 0, 0)),
            pl.BlockSpec((1, F, H), lambda b, s, sa: (sa[b], 0, 0)),
            pl.BlockSpec((1, 1, H), lambda b, s, sa: (sa[b], 0, 0)),
            pl.BlockSpec(memory_space=pltpu.MemorySpace.VMEM),
            pl.BlockSpec(memory_space=pltpu.MemorySpace.VMEM),
        ],
        out_specs=pl.BlockSpec((1, TS, H), lambda b, s, sa: (b, s, 0)),
    )
    out = pl.pallas_call(
        _moe_kernel,
        grid_spec=grid_spec,
        out_shape=jax.ShapeDtypeStruct((B, S, H), jnp.float32),
        compiler_params=pltpu.CompilerParams(
            dimension_semantics=("arbitrary", "arbitrary")),
    )(act16, hidden_states, bW1, bb1.reshape(NB, 1, F), bW2,
      bb2.reshape(NB, 1, H), oW, ob.reshape(1, -1))

    return (out, probs, selected, value)


# speedup vs baseline: 1.0517x; 1.0014x over previous
import functools

import jax
import jax.numpy as jnp
from jax import lax
from jax.experimental import pallas as pl
from jax.experimental.pallas import tpu as pltpu
from jax.experimental.pallas import tpu_sc as plsc


def _gelu(x):
    return 0.5 * x * (1.0 + jax.lax.erf(x * 0.7071067811865476))


def _nav_kernel(last_ref, pW1_ref, pb1_ref, pW2T_ref, pb2T_ref,
                vW1_ref, vb1_ref, vW2_ref, vb2_ref,
                lt_ref, val_ref):
    h = last_ref[...]
    B = h.shape[0]
    g1 = _gelu(jnp.dot(h, pW1_ref[...],
                       preferred_element_type=jnp.float32) + pb1_ref[...])
    logits_t = lax.dot_general(
        pW2T_ref[...], g1, (((1,), (1,)), ((), ())),
        preferred_element_type=jnp.float32) + pb2T_ref[...]
    lt_ref[...] = jnp.concatenate(
        [logits_t, jnp.zeros((logits_t.shape[0], 16 - B), jnp.float32)],
        axis=1)
    v1 = _gelu(jnp.dot(h, vW1_ref[...],
                       preferred_element_type=jnp.float32) + vb1_ref[...])
    v = jnp.dot(v1, vW2_ref[...],
                preferred_element_type=jnp.float32) + vb2_ref[...]
    val_ref[...] = v.T


def _sc_router_kernel(lt_hbm, probs_hbm, act_hbm, lt_v, probs_s, act_s):
    wid = lax.axis_index("c") * 16 + lax.axis_index("s")

    pltpu.sync_copy(lt_hbm, lt_v)
    l0 = lt_v[0, :]
    l1 = lt_v[1, :]
    l2 = lt_v[2, :]
    l3 = lt_v[3, :]
    m = jnp.maximum(jnp.maximum(l0, l1), jnp.maximum(l2, l3))
    e0 = jnp.exp(l0 - m)
    e1 = jnp.exp(l1 - m)
    e2 = jnp.exp(l2 - m)
    e3 = jnp.exp(l3 - m)
    s = e0 + e1 + e2 + e3
    probs_s[0, :] = e0 / s
    probs_s[1, :] = e1 / s
    probs_s[2, :] = e2 / s
    probs_s[3, :] = e3 / s
    am = jnp.full((16,), 3, jnp.int32)
    am = jnp.where(l2 == m, jnp.full((16,), 2, jnp.int32), am)
    am = jnp.where(l1 == m, jnp.full((16,), 1, jnp.int32), am)
    am = jnp.where(l0 == m, jnp.full((16,), 0, jnp.int32), am)
    act_s[...] = am

    @pl.when(wid == 0)
    def _():
        pltpu.sync_copy(probs_s, probs_hbm)
        pltpu.sync_copy(act_s, act_hbm)


def _moe_kernel(sa_ref, x_ref, w1_ref, b1_ref, w2_ref, b2_ref,
                ow_ref, ob_ref, out_ref):
    del sa_ref
    TS = x_ref.shape[1]
    F = w1_ref.shape[2]
    FC = F
    RC = TS // 2
    for r in range(2):
        x = x_ref[0, pl.ds(r * RC, RC)]
        y = b2_ref[0].astype(jnp.float32)
        for c in range(1):
            sl = slice(c * FC, (c + 1) * FC)
            midc = _gelu(jnp.dot(x, w1_ref[0, :, sl],
                                 preferred_element_type=jnp.float32)
                         + b1_ref[0, :, sl])
            y = y + jnp.dot(midc, w2_ref[0, sl, :],
                            preferred_element_type=jnp.float32)
        z = jnp.dot(y, ow_ref[...],
                    preferred_element_type=jnp.float32) + ob_ref[...]
        out_ref[0, pl.ds(r * RC, RC)] = x + 0.3 * z


def kernel(hidden_states, pW1, pb1, pW2, pb2, vW1, vb1, vW2, vb2,
           bW1, bb1, bW2, bb2, oW, ob):
    B, S, H = hidden_states.shape
    NB = pW2.shape[1]
    F = bW1.shape[2]
    TS = 1024

    last = hidden_states[:, -1, :]
    lt16, val2 = pl.pallas_call(
        _nav_kernel,
        out_shape=[
            jax.ShapeDtypeStruct((NB, 16), jnp.float32),
            jax.ShapeDtypeStruct((1, B), jnp.float32),
        ],
    )(last, pW1, pb1.reshape(1, -1), pW2.T, pb2.reshape(-1, 1),
      vW1, vb1.reshape(1, -1), vW2, vb2.reshape(1, 1))
    value = val2[0]

    mesh = plsc.VectorSubcoreMesh(core_axis_name="c", subcore_axis_name="s")
    sc_router = functools.partial(
        pl.kernel,
        mesh=mesh,
        out_type=[
            jax.ShapeDtypeStruct((NB, 16), jnp.float32),
            jax.ShapeDtypeStruct((16,), jnp.int32),
        ],
        scratch_types=[
            pltpu.VMEM((NB, 16), jnp.float32),
            pltpu.VMEM((NB, 16), jnp.float32),
            pltpu.VMEM((16,), jnp.int32),
        ],
    )(_sc_router_kernel)
    probs16, act16 = sc_router(lt16)
    probs = probs16[:, :B].T
    selected = act16[:B]

    grid_spec = pltpu.PrefetchScalarGridSpec(
        num_scalar_prefetch=1,
        grid=(B, S // TS),
        in_specs=[
            pl.BlockSpec((1, TS, H), lambda b, s, sa: (b, s, 0)),
            pl.BlockSpec((1, H, F), lambda b, s, sa: (sa[b], 0, 0)),
            pl.BlockSpec((1, 1, F), lambda b, s, sa: (sa[b], 0, 0)),
            pl.BlockSpec((1, F, H), lambda b, s, sa: (sa[b], 0, 0)),
            pl.BlockSpec((1, 1, H), lambda b, s, sa: (sa[b], 0, 0)),
            pl.BlockSpec(memory_space=pltpu.MemorySpace.VMEM),
            pl.BlockSpec(memory_space=pltpu.MemorySpace.VMEM),
        ],
        out_specs=pl.BlockSpec((1, TS, H), lambda b, s, sa: (b, s, 0)),
    )
    out = pl.pallas_call(
        _moe_kernel,
        grid_spec=grid_spec,
        out_shape=jax.ShapeDtypeStruct((B, S, H), jnp.float32),
        compiler_params=pltpu.CompilerParams(
            dimension_semantics=("arbitrary", "arbitrary")),
    )(act16, hidden_states, bW1, bb1.reshape(NB, 1, F), bW2,
      bb2.reshape(NB, 1, H), oW, ob.reshape(1, -1))

    return (out, probs, selected, value)


# speedup vs baseline: 1.0545x; 1.0027x over previous
import functools

import jax
import jax.numpy as jnp
from jax import lax
from jax.experimental import pallas as pl
from jax.experimental.pallas import tpu as pltpu
from jax.experimental.pallas import tpu_sc as plsc


def _gelu(x):
    return 0.5 * x * (1.0 + jax.lax.erf(x * 0.7071067811865476))


def _nav_kernel(last_ref, pW1_ref, pb1_ref, pW2T_ref, pb2T_ref,
                vW1_ref, vb1_ref, vW2_ref, vb2_ref,
                lt_ref, val_ref):
    h = last_ref[...]
    B = h.shape[0]
    g1 = _gelu(jnp.dot(h, pW1_ref[...],
                       preferred_element_type=jnp.float32) + pb1_ref[...])
    logits_t = lax.dot_general(
        pW2T_ref[...], g1, (((1,), (1,)), ((), ())),
        preferred_element_type=jnp.float32) + pb2T_ref[...]
    lt_ref[...] = jnp.concatenate(
        [logits_t, jnp.zeros((logits_t.shape[0], 16 - B), jnp.float32)],
        axis=1)
    v1 = _gelu(jnp.dot(h, vW1_ref[...],
                       preferred_element_type=jnp.float32) + vb1_ref[...])
    v = jnp.dot(v1, vW2_ref[...],
                preferred_element_type=jnp.float32) + vb2_ref[...]
    val_ref[...] = v.T


def _sc_router_kernel(lt_hbm, probs_hbm, act_hbm, lt_v, probs_s, act_s):
    wid = lax.axis_index("c") * 16 + lax.axis_index("s")

    pltpu.sync_copy(lt_hbm, lt_v)
    l0 = lt_v[0, :]
    l1 = lt_v[1, :]
    l2 = lt_v[2, :]
    l3 = lt_v[3, :]
    m = jnp.maximum(jnp.maximum(l0, l1), jnp.maximum(l2, l3))
    e0 = jnp.exp(l0 - m)
    e1 = jnp.exp(l1 - m)
    e2 = jnp.exp(l2 - m)
    e3 = jnp.exp(l3 - m)
    s = e0 + e1 + e2 + e3
    probs_s[0, :] = e0 / s
    probs_s[1, :] = e1 / s
    probs_s[2, :] = e2 / s
    probs_s[3, :] = e3 / s
    am = jnp.full((16,), 3, jnp.int32)
    am = jnp.where(l2 == m, jnp.full((16,), 2, jnp.int32), am)
    am = jnp.where(l1 == m, jnp.full((16,), 1, jnp.int32), am)
    am = jnp.where(l0 == m, jnp.full((16,), 0, jnp.int32), am)
    act_s[...] = am

    @pl.when(wid == 0)
    def _():
        pltpu.sync_copy(probs_s, probs_hbm)
        pltpu.sync_copy(act_s, act_hbm)


def _moe_kernel(sa_ref, x_ref, w1_ref, b1_ref, w2_ref, b2_ref,
                ow_ref, ob_ref, out_ref):
    del sa_ref
    TS = x_ref.shape[1]
    F = w1_ref.shape[2]
    FC = F
    RC = TS // 4
    for r in range(4):
        x = x_ref[0, pl.ds(r * RC, RC)]
        y = b2_ref[0].astype(jnp.float32)
        for c in range(1):
            sl = slice(c * FC, (c + 1) * FC)
            midc = _gelu(jnp.dot(x, w1_ref[0, :, sl],
                                 preferred_element_type=jnp.float32)
                         + b1_ref[0, :, sl])
            y = y + jnp.dot(midc, w2_ref[0, sl, :],
                            preferred_element_type=jnp.float32)
        z = jnp.dot(y, ow_ref[...],
                    preferred_element_type=jnp.float32) + ob_ref[...]
        out_ref[0, pl.ds(r * RC, RC)] = x + 0.3 * z


def kernel(hidden_states, pW1, pb1, pW2, pb2, vW1, vb1, vW2, vb2,
           bW1, bb1, bW2, bb2, oW, ob):
    B, S, H = hidden_states.shape
    NB = pW2.shape[1]
    F = bW1.shape[2]
    TS = 1024

    last = hidden_states[:, -1, :]
    lt16, val2 = pl.pallas_call(
        _nav_kernel,
        out_shape=[
            jax.ShapeDtypeStruct((NB, 16), jnp.float32),
            jax.ShapeDtypeStruct((1, B), jnp.float32),
        ],
    )(last, pW1, pb1.reshape(1, -1), pW2.T, pb2.reshape(-1, 1),
      vW1, vb1.reshape(1, -1), vW2, vb2.reshape(1, 1))
    value = val2[0]

    mesh = plsc.VectorSubcoreMesh(core_axis_name="c", subcore_axis_name="s")
    sc_router = functools.partial(
        pl.kernel,
        mesh=mesh,
        out_type=[
            jax.ShapeDtypeStruct((NB, 16), jnp.float32),
            jax.ShapeDtypeStruct((16,), jnp.int32),
        ],
        scratch_types=[
            pltpu.VMEM((NB, 16), jnp.float32),
            pltpu.VMEM((NB, 16), jnp.float32),
            pltpu.VMEM((16,), jnp.int32),
        ],
    )(_sc_router_kernel)
    probs16, act16 = sc_router(lt16)
    probs = probs16[:, :B].T
    selected = act16[:B]

    grid_spec = pltpu.PrefetchScalarGridSpec(
        num_scalar_prefetch=1,
        grid=(B, S // TS),
        in_specs=[
            pl.BlockSpec((1, TS, H), lambda b, s, sa: (b, s, 0)),
            pl.BlockSpec((1, H, F), lambda b, s, sa: (sa[b], 0, 0)),
            pl.BlockSpec((1, 1, F), lambda b, s, sa: (sa[b], 0, 0)),
            pl.BlockSpec((1, F, H), lambda b, s, sa: (sa[b], 0, 0)),
            pl.BlockSpec((1, 1, H), lambda b, s, sa: (sa[b], 0, 0)),
            pl.BlockSpec(memory_space=pltpu.MemorySpace.VMEM),
            pl.BlockSpec(memory_space=pltpu.MemorySpace.VMEM),
        ],
        out_specs=pl.BlockSpec((1, TS, H), lambda b, s, sa: (b, s, 0)),
    )
    out = pl.pallas_call(
        _moe_kernel,
        grid_spec=grid_spec,
        out_shape=jax.ShapeDtypeStruct((B, S, H), jnp.float32),
        compiler_params=pltpu.CompilerParams(
            dimension_semantics=("arbitrary", "arbitrary")),
    )(act16, hidden_states, bW1, bb1.reshape(NB, 1, F), bW2,
      bb2.reshape(NB, 1, H), oW, ob.reshape(1, -1))

    return (out, probs, selected, value)
